# baseline (speedup 1.0000x reference)
import functools

import jax
import jax.numpy as jnp
from jax import lax
from jax.experimental import pallas as pl
from jax.experimental.pallas import tpu as pltpu
from jax.experimental.pallas import tpu_sc as plsc

B = 4096
L = 200
D = 64
H = 128
O = 10
HALF = L // 2
NVREG = D // 16
WPC = 4


def _pool_sc(text2, emb2):
    info = plsc.get_sparse_core_info()
    ncores = info.num_cores
    nw = ncores * info.num_subcores
    rpw = B // nw
    nchunk = 2 * rpw
    mesh = plsc.VectorSubcoreMesh(core_axis_name="c", subcore_axis_name="s")

    @functools.partial(
        pl.kernel,
        out_type=jax.ShapeDtypeStruct((B, D), jnp.float32),
        mesh=mesh,
        scratch_types=[
            pltpu.VMEM((nchunk, HALF), jnp.int32),
            pltpu.VMEM((L, 2 * D), jnp.float32),
            pltpu.VMEM((L, 2 * D), jnp.float32),
            pltpu.VMEM((rpw, D), jnp.float32),
            pltpu.SemaphoreType.DMA,
            pltpu.SemaphoreType.DMA,
        ],
    )
    def pool(text_hbm, emb_hbm, out_hbm,
             gidx_v, rows0, rows1, out_v, sem0, sem1):
        wid = lax.axis_index("s") * ncores + lax.axis_index("c")
        base = wid * rpw
        pltpu.sync_copy(text_hbm.at[pl.ds(2 * base, nchunk)], gidx_v)
        bufs = (rows0, rows1)
        sems = (sem0, sem1)

        def issue(b, t):
            pltpu.async_copy(emb_hbm.at[gidx_v.at[2 * b]],
                             bufs[t].at[pl.ds(0, HALF)], sems[t])
            pltpu.async_copy(emb_hbm.at[gidx_v.at[2 * b + 1]],
                             bufs[t].at[pl.ds(HALF, HALF)], sems[t])

        def drain(t):
            pltpu.make_async_copy(emb_hbm.at[pl.ds(0, L)], bufs[t], sems[t]).wait()

        def consume(b, t):
            drain(t)
            buf = bufs[t]
            zero = jnp.zeros((16,), jnp.float32)

            def rbody(r, acc):
                return tuple(acc[d] + buf[r, pl.ds(d * 16, 16)]
                             for d in range(NVREG))

            acc = lax.fori_loop(0, L, rbody, (zero,) * NVREG, unroll=8)
            inv = jnp.float32(1.0 / L)
            for d in range(NVREG):
                out_v[b, pl.ds(d * 16, 16)] = acc[d] * inv

        issue(0, 0)

        def outer(i, carry):
            for t in range(2):
                b = 2 * i + t

                @pl.when(b + 1 < rpw)
                def _():
                    issue(b + 1, (t + 1) % 2)

                consume(b, t)
            return carry

        lax.fori_loop(0, rpw // 2, outer, 0)
        pltpu.sync_copy(out_v, out_hbm.at[pl.ds(base, rpw)])

    return pool(text2, emb2)


def _tr_body(x_ref, o_ref):
    xt = lax.dot_general(x_ref[...], jnp.eye(D, dtype=jnp.float32),
                         (((0,), (0,)), ((), ())),
                         precision=lax.Precision.HIGHEST,
                         preferred_element_type=jnp.float32)
    o_ref[:, 0:D] = xt
    o_ref[:, D:2 * D] = xt


def _transpose_tc(embT):
    V = embT.shape[1]
    K = 4096
    grid = (V + K - 1) // K
    return pl.pallas_call(
        _tr_body,
        grid=(grid,),
        in_specs=[pl.BlockSpec((D, K), lambda i: (0, i))],
        out_specs=pl.BlockSpec((K, 2 * D), lambda i: (i, 0)),
        out_shape=jax.ShapeDtypeStruct((V, 2 * D), jnp.float32),
    )(embT)


def _mlp_body(x_ref, w1_ref, b1_ref, w2_ref, b2_ref, o_ref):
    x = x_ref[...]
    h = lax.dot_general(x, w1_ref[...], (((1,), (1,)), ((), ())),
                        preferred_element_type=jnp.float32)
    h = jnp.maximum(h + b1_ref[...], 0.0)
    o = lax.dot_general(h, w2_ref[...], (((1,), (1,)), ((), ())),
                        preferred_element_type=jnp.float32)
    o_ref[...] = o + b2_ref[...]


def _mlp_tc(pooled, W1, b1, W2, b2):
    blk = 512
    return pl.pallas_call(
        _mlp_body,
        grid=(B // blk,),
        in_specs=[
            pl.BlockSpec((blk, D), lambda i: (i, 0)),
            pl.BlockSpec((H, D), lambda i: (0, 0)),
            pl.BlockSpec((1, H), lambda i: (0, 0)),
            pl.BlockSpec((O, H), lambda i: (0, 0)),
            pl.BlockSpec((1, O), lambda i: (0, 0)),
        ],
        out_specs=pl.BlockSpec((blk, O), lambda i: (i, 0)),
        out_shape=jax.ShapeDtypeStruct((B, O), jnp.float32),
    )(pooled, W1, b1.reshape(1, H), W2, b2.reshape(1, O))


def kernel(text, emb, W1, b1, W2, b2):
    text2 = text.astype(jnp.int32).reshape(2 * B, HALF)
    emb2 = _transpose_tc(emb.T)
    pooled = _pool_sc(text2, emb2)
    return _mlp_tc(pooled, W1, b1, W2, b2)


# speedup vs baseline: 1.1619x; 1.1619x over previous
import functools

import jax
import jax.numpy as jnp
from jax import lax
from jax.experimental import pallas as pl
from jax.experimental.pallas import tpu as pltpu
from jax.experimental.pallas import tpu_sc as plsc

B = 4096
L = 200
D = 64
H = 128
O = 10
HALF = L // 2
NVREG = D // 16
WPC = 4


def _pool_sc(text2, emb2):
    info = plsc.get_sparse_core_info()
    ncores = info.num_cores
    nw = ncores * info.num_subcores
    rpw = B // nw
    nchunk = 2 * rpw
    mesh = plsc.VectorSubcoreMesh(core_axis_name="c", subcore_axis_name="s")

    @functools.partial(
        pl.kernel,
        out_type=jax.ShapeDtypeStruct((B, D), jnp.float32),
        mesh=mesh,
        scratch_types=[
            pltpu.VMEM((nchunk, HALF), jnp.int32),
            pltpu.VMEM((L, 2 * D), jnp.float32),
            pltpu.VMEM((L, 2 * D), jnp.float32),
            pltpu.VMEM((rpw, D), jnp.float32),
            pltpu.SemaphoreType.DMA,
            pltpu.SemaphoreType.DMA,
        ],
    )
    def pool(text_hbm, emb_hbm, out_hbm,
             gidx_v, rows0, rows1, out_v, sem0, sem1):
        wid = lax.axis_index("s") * ncores + lax.axis_index("c")
        base = wid * rpw
        pltpu.sync_copy(text_hbm.at[pl.ds(2 * base, nchunk)], gidx_v)
        bufs = (rows0, rows1)
        sems = (sem0, sem1)

        def issue(b, t):
            pltpu.async_copy(emb_hbm.at[gidx_v.at[2 * b]],
                             bufs[t].at[pl.ds(0, HALF)], sems[t])
            pltpu.async_copy(emb_hbm.at[gidx_v.at[2 * b + 1]],
                             bufs[t].at[pl.ds(HALF, HALF)], sems[t])

        def drain(t):
            pltpu.make_async_copy(emb_hbm.at[pl.ds(0, L)], bufs[t], sems[t]).wait()

        def consume(b, t):
            drain(t)
            buf = bufs[t]
            zero = jnp.zeros((16,), jnp.float32)

            def rbody(r, acc):
                return tuple(acc[d] + buf[r, pl.ds(d * 16, 16)]
                             for d in range(NVREG))

            acc = lax.fori_loop(0, L, rbody, (zero,) * NVREG, unroll=8)
            inv = jnp.float32(1.0 / L)
            for d in range(NVREG):
                out_v[b, pl.ds(d * 16, 16)] = acc[d] * inv

        issue(0, 0)

        def outer(i, carry):
            for t in range(2):
                b = 2 * i + t

                @pl.when(b + 1 < rpw)
                def _():
                    issue(b + 1, (t + 1) % 2)

                consume(b, t)
            return carry

        lax.fori_loop(0, rpw // 2, outer, 0)
        pltpu.sync_copy(out_v, out_hbm.at[pl.ds(base, rpw)])

    return pool(text2, emb2)


def _tr_body(x_ref, o_ref):
    xt = x_ref[...].T
    o_ref[:, 0:D] = xt
    o_ref[:, D:2 * D] = xt


def _transpose_tc(embT):
    V = embT.shape[1]
    K = 4096
    grid = (V + K - 1) // K
    return pl.pallas_call(
        _tr_body,
        grid=(grid,),
        in_specs=[pl.BlockSpec((D, K), lambda i: (0, i))],
        out_specs=pl.BlockSpec((K, 2 * D), lambda i: (i, 0)),
        out_shape=jax.ShapeDtypeStruct((V, 2 * D), jnp.float32),
    )(embT)


def _mlp_body(x_ref, w1_ref, b1_ref, w2_ref, b2_ref, o_ref):
    x = x_ref[...]
    h = lax.dot_general(x, w1_ref[...], (((1,), (1,)), ((), ())),
                        preferred_element_type=jnp.float32)
    h = jnp.maximum(h + b1_ref[...], 0.0)
    o = lax.dot_general(h, w2_ref[...], (((1,), (1,)), ((), ())),
                        preferred_element_type=jnp.float32)
    o_ref[...] = o + b2_ref[...]


def _mlp_tc(pooled, W1, b1, W2, b2):
    blk = 512
    return pl.pallas_call(
        _mlp_body,
        grid=(B // blk,),
        in_specs=[
            pl.BlockSpec((blk, D), lambda i: (i, 0)),
            pl.BlockSpec((H, D), lambda i: (0, 0)),
            pl.BlockSpec((1, H), lambda i: (0, 0)),
            pl.BlockSpec((O, H), lambda i: (0, 0)),
            pl.BlockSpec((1, O), lambda i: (0, 0)),
        ],
        out_specs=pl.BlockSpec((blk, O), lambda i: (i, 0)),
        out_shape=jax.ShapeDtypeStruct((B, O), jnp.float32),
    )(pooled, W1, b1.reshape(1, H), W2, b2.reshape(1, O))


def kernel(text, emb, W1, b1, W2, b2):
    text2 = text.astype(jnp.int32).reshape(2 * B, HALF)
    emb2 = _transpose_tc(emb.T)
    pooled = _pool_sc(text2, emb2)
    return _mlp_tc(pooled, W1, b1, W2, b2)


# speedup vs baseline: 1.2521x; 1.0776x over previous
import functools

import jax
import jax.numpy as jnp
from jax import lax
from jax.experimental import pallas as pl
from jax.experimental.pallas import tpu as pltpu
from jax.experimental.pallas import tpu_sc as plsc

B = 4096
L = 200
D = 64
H = 128
O = 10
HALF = L // 2
NVREG = D // 16
WPC = 4


def _pool_sc(gtext, hbits, emb2):
    info = plsc.get_sparse_core_info()
    ncores = info.num_cores
    nw = ncores * info.num_subcores
    rpw = B // nw
    nchunk = 2 * rpw
    mesh = plsc.VectorSubcoreMesh(core_axis_name="c", subcore_axis_name="s")

    @functools.partial(
        pl.kernel,
        out_type=jax.ShapeDtypeStruct((B, D), jnp.float32),
        mesh=mesh,
        scratch_types=[
            pltpu.VMEM((nchunk, HALF), jnp.int32),
            pltpu.VMEM((nchunk * WPC + 16,), jnp.int32),
            pltpu.VMEM((L, 2 * D), jnp.float32),
            pltpu.VMEM((L, 2 * D), jnp.float32),
            pltpu.VMEM((rpw, D), jnp.float32),
            pltpu.SemaphoreType.DMA,
            pltpu.SemaphoreType.DMA,
        ],
    )
    def pool(gtext_hbm, hbits_hbm, emb_hbm, out_hbm,
             gidx_v, bits_v, rows0, rows1, out_v, sem0, sem1):
        wid = lax.axis_index("s") * ncores + lax.axis_index("c")
        base = wid * rpw
        pltpu.sync_copy(gtext_hbm.at[pl.ds(2 * base, nchunk)], gidx_v)
        pltpu.sync_copy(hbits_hbm.at[pl.ds(2 * base * WPC, nchunk * WPC)],
                        bits_v.at[pl.ds(0, nchunk * WPC)])
        bufs = (rows0, rows1)
        sems = (sem0, sem1)

        def issue(b, t):
            pltpu.async_copy(emb_hbm.at[gidx_v.at[2 * b]],
                             bufs[t].at[pl.ds(0, HALF)], sems[t])
            pltpu.async_copy(emb_hbm.at[gidx_v.at[2 * b + 1]],
                             bufs[t].at[pl.ds(HALF, HALF)], sems[t])

        def drain(t):
            pltpu.make_async_copy(emb_hbm.at[pl.ds(0, L)], bufs[t], sems[t]).wait()

        def consume(b, t):
            drain(t)
            buf = bufs[t]
            zero = jnp.zeros((16,), jnp.float32)

            def make_rbody(h):
                def rbody(rr, acc):
                    w = bits_v[pl.ds((2 * b + h) * WPC + (rr >> 5), 16)][0]
                    off = (lax.shift_right_logical(w, rr & 31) & 1) * D
                    r = h * HALF + rr
                    return tuple(acc[d] + buf[r, pl.ds(off + d * 16, 16)]
                                 for d in range(NVREG))
                return rbody

            acc = lax.fori_loop(0, HALF, make_rbody(0), (zero,) * NVREG,
                                unroll=8)
            acc = lax.fori_loop(0, HALF, make_rbody(1), acc, unroll=8)
            inv = jnp.float32(1.0 / L)
            for d in range(NVREG):
                out_v[b, pl.ds(d * 16, 16)] = acc[d] * inv

        issue(0, 0)

        def outer(i, carry):
            for t in range(2):
                b = 2 * i + t

                @pl.when(b + 1 < rpw)
                def _():
                    issue(b + 1, (t + 1) % 2)

                consume(b, t)
            return carry

        lax.fori_loop(0, rpw // 2, outer, 0)
        pltpu.sync_copy(out_v, out_hbm.at[pl.ds(base, rpw)])

    return pool(gtext, hbits, emb2)


KTR = 2048
NTR = 245
M = KTR * NTR


def _tr_body(xa_ref, xb_ref, o_ref):
    o_ref[:, 0:D] = xa_ref[...].T
    o_ref[:, D:2 * D] = xb_ref[...].T


def _transpose_tc(embT):
    return pl.pallas_call(
        _tr_body,
        grid=(NTR,),
        in_specs=[pl.BlockSpec((D, KTR), lambda i: (0, i)),
                  pl.BlockSpec((D, KTR),
                               lambda i: (0, jnp.minimum(i + NTR, 488)))],
        out_specs=pl.BlockSpec((KTR, 2 * D), lambda i: (i, 0)),
        out_shape=jax.ShapeDtypeStruct((M, 2 * D), jnp.float32),
    )(embT, embT)


def _mlp_body(x_ref, w1_ref, b1_ref, w2_ref, b2_ref, o_ref):
    x = x_ref[...]
    h = lax.dot_general(x, w1_ref[...], (((1,), (1,)), ((), ())),
                        preferred_element_type=jnp.float32)
    h = jnp.maximum(h + b1_ref[...], 0.0)
    o = lax.dot_general(h, w2_ref[...], (((1,), (1,)), ((), ())),
                        preferred_element_type=jnp.float32)
    o_ref[...] = o + b2_ref[...]


def _mlp_tc(pooled, W1, b1, W2, b2):
    blk = 512
    return pl.pallas_call(
        _mlp_body,
        grid=(B // blk,),
        in_specs=[
            pl.BlockSpec((blk, D), lambda i: (i, 0)),
            pl.BlockSpec((H, D), lambda i: (0, 0)),
            pl.BlockSpec((1, H), lambda i: (0, 0)),
            pl.BlockSpec((O, H), lambda i: (0, 0)),
            pl.BlockSpec((1, O), lambda i: (0, 0)),
        ],
        out_specs=pl.BlockSpec((blk, O), lambda i: (i, 0)),
        out_shape=jax.ShapeDtypeStruct((B, O), jnp.float32),
    )(pooled, W1, b1.reshape(1, H), W2, b2.reshape(1, O))


def kernel(text, emb, W1, b1, W2, b2):
    ti = text.astype(jnp.int32)
    hi = (ti >= M).astype(jnp.int32)
    gtext = (ti - hi * M).reshape(2 * B, HALF)
    par = hi.reshape(2 * B, HALF)
    parp = jnp.pad(par, ((0, 0), (0, 28))).reshape(2 * B, WPC, 32)
    shifts = jnp.arange(32, dtype=jnp.int32)[None, None, :]
    hbits = (parp << shifts).sum(axis=2, dtype=jnp.int32).reshape(-1)
    emb2 = _transpose_tc(emb.T)
    pooled = _pool_sc(gtext, hbits, emb2)
    return _mlp_tc(pooled, W1, b1, W2, b2)


# speedup vs baseline: 1.4180x; 1.1325x over previous
import functools

import jax
import jax.numpy as jnp
from jax import lax
from jax.experimental import pallas as pl
from jax.experimental.pallas import tpu as pltpu
from jax.experimental.pallas import tpu_sc as plsc

B = 4096
L = 200
D = 64
H = 128
O = 10
HALF = L // 2
NVREG = D // 16
WPC = 4


def _pool_sc(gtext, hbits, emb2):
    info = plsc.get_sparse_core_info()
    ncores = info.num_cores
    nw = ncores * info.num_subcores
    rpw = B // nw
    nchunk = 2 * rpw
    mesh = plsc.VectorSubcoreMesh(core_axis_name="c", subcore_axis_name="s")

    @functools.partial(
        pl.kernel,
        out_type=jax.ShapeDtypeStruct((B, D), jnp.float32),
        mesh=mesh,
        scratch_types=[
            pltpu.VMEM((nchunk, HALF), jnp.int32),
            pltpu.VMEM((nchunk * WPC + 16,), jnp.int32),
            pltpu.VMEM((L, 2 * D), jnp.float32),
            pltpu.VMEM((L, 2 * D), jnp.float32),
            pltpu.VMEM((rpw, D), jnp.float32),
            pltpu.SemaphoreType.DMA,
            pltpu.SemaphoreType.DMA,
        ],
    )
    def pool(gtext_hbm, hbits_hbm, emb_hbm, out_hbm,
             gidx_v, bits_v, rows0, rows1, out_v, sem0, sem1):
        wid = lax.axis_index("s") * ncores + lax.axis_index("c")
        base = wid * rpw
        pltpu.sync_copy(gtext_hbm.at[pl.ds(2 * base, nchunk)], gidx_v)
        pltpu.sync_copy(hbits_hbm.at[pl.ds(2 * base * WPC, nchunk * WPC)],
                        bits_v.at[pl.ds(0, nchunk * WPC)])
        bufs = (rows0, rows1)
        sems = (sem0, sem1)

        def issue(b, t):
            pltpu.async_copy(emb_hbm.at[gidx_v.at[2 * b]],
                             bufs[t].at[pl.ds(0, HALF)], sems[t])
            pltpu.async_copy(emb_hbm.at[gidx_v.at[2 * b + 1]],
                             bufs[t].at[pl.ds(HALF, HALF)], sems[t])

        def drain(t):
            pltpu.make_async_copy(emb_hbm.at[pl.ds(0, L)], bufs[t], sems[t]).wait()

        def consume(b, t):
            drain(t)
            buf = bufs[t]
            zero = jnp.zeros((16,), jnp.float32)

            def make_rbody(h):
                def rbody(rr, acc):
                    w = bits_v[pl.ds((2 * b + h) * WPC + (rr >> 5), 16)][0]
                    off = (lax.shift_right_logical(w, rr & 31) & 1) * D
                    r = h * HALF + rr
                    return tuple(acc[d] + buf[r, pl.ds(off + d * 16, 16)]
                                 for d in range(NVREG))
                return rbody

            acc = lax.fori_loop(0, HALF, make_rbody(0), (zero,) * NVREG,
                                unroll=8)
            acc = lax.fori_loop(0, HALF, make_rbody(1), acc, unroll=8)
            inv = jnp.float32(1.0 / L)
            for d in range(NVREG):
                out_v[b, pl.ds(d * 16, 16)] = acc[d] * inv

        issue(0, 0)

        def outer(i, carry):
            for t in range(2):
                b = 2 * i + t

                @pl.when(b + 1 < rpw)
                def _():
                    issue(b + 1, (t + 1) % 2)

                consume(b, t)
            return carry

        lax.fori_loop(0, rpw // 2, outer, 0)
        pltpu.sync_copy(out_v, out_hbm.at[pl.ds(base, rpw)])

    return pool(gtext, hbits, emb2)


KTR = 4096
NTR = 123
M = KTR * NTR


def _tr_body(xa_ref, xb_ref, o_ref):
    o_ref[:, 0:D] = xa_ref[...].T
    o_ref[:, D:2 * D] = xb_ref[...].T


def _transpose_tc(embT):
    return pl.pallas_call(
        _tr_body,
        grid=(NTR,),
        in_specs=[pl.BlockSpec((D, KTR), lambda i: (0, i)),
                  pl.BlockSpec((D, KTR),
                               lambda i: (0, jnp.minimum(i + NTR, 244)))],
        out_specs=pl.BlockSpec((KTR, 2 * D), lambda i: (i, 0)),
        out_shape=jax.ShapeDtypeStruct((M, 2 * D), jnp.float32),
    )(embT, embT)


def _mlp_body(x_ref, w1_ref, b1_ref, w2_ref, b2_ref, o_ref):
    x = x_ref[...]
    h = lax.dot_general(x, w1_ref[...], (((1,), (1,)), ((), ())),
                        preferred_element_type=jnp.float32)
    h = jnp.maximum(h + b1_ref[...], 0.0)
    o = lax.dot_general(h, w2_ref[...], (((1,), (1,)), ((), ())),
                        preferred_element_type=jnp.float32)
    o_ref[...] = o + b2_ref[...]


def _mlp_tc(pooled, W1, b1, W2, b2):
    blk = 512
    return pl.pallas_call(
        _mlp_body,
        grid=(B // blk,),
        in_specs=[
            pl.BlockSpec((blk, D), lambda i: (i, 0)),
            pl.BlockSpec((H, D), lambda i: (0, 0)),
            pl.BlockSpec((1, H), lambda i: (0, 0)),
            pl.BlockSpec((O, H), lambda i: (0, 0)),
            pl.BlockSpec((1, O), lambda i: (0, 0)),
        ],
        out_specs=pl.BlockSpec((blk, O), lambda i: (i, 0)),
        out_shape=jax.ShapeDtypeStruct((B, O), jnp.float32),
    )(pooled, W1, b1.reshape(1, H), W2, b2.reshape(1, O))


def kernel(text, emb, W1, b1, W2, b2):
    ti = text.astype(jnp.int32)
    hi = (ti >= M).astype(jnp.int32)
    gtext = (ti - hi * M).reshape(2 * B, HALF)
    par = hi.reshape(2 * B, HALF)
    parp = jnp.pad(par, ((0, 0), (0, 28))).reshape(2 * B, WPC, 32)
    shifts = jnp.arange(32, dtype=jnp.int32)[None, None, :]
    hbits = (parp << shifts).sum(axis=2, dtype=jnp.int32).reshape(-1)
    emb2 = _transpose_tc(emb.T)
    pooled = _pool_sc(gtext, hbits, emb2)
    return _mlp_tc(pooled, W1, b1, W2, b2)


# speedup vs baseline: 1.5173x; 1.0700x over previous
import functools

import jax
import jax.numpy as jnp
from jax import lax
from jax.experimental import pallas as pl
from jax.experimental.pallas import tpu as pltpu
from jax.experimental.pallas import tpu_sc as plsc

B = 4096
L = 200
D = 64
H = 128
O = 10
HALF = L // 2
NVREG = D // 16
WPC = 4


def _pool_sc(gtext, hbits, emb2):
    info = plsc.get_sparse_core_info()
    ncores = info.num_cores
    nw = ncores * info.num_subcores
    rpw = B // nw
    nchunk = 2 * rpw
    mesh = plsc.VectorSubcoreMesh(core_axis_name="c", subcore_axis_name="s")

    @functools.partial(
        pl.kernel,
        out_type=jax.ShapeDtypeStruct((B, D), jnp.float32),
        mesh=mesh,
        scratch_types=[
            pltpu.VMEM((nchunk, HALF), jnp.int32),
            pltpu.VMEM((nchunk * WPC + 16,), jnp.int32),
            pltpu.VMEM((L, 2 * D), jnp.float32),
            pltpu.VMEM((L, 2 * D), jnp.float32),
            pltpu.VMEM((rpw, D), jnp.float32),
            pltpu.SemaphoreType.DMA,
            pltpu.SemaphoreType.DMA,
        ],
    )
    def pool(gtext_hbm, hbits_hbm, emb_hbm, out_hbm,
             gidx_v, bits_v, rows0, rows1, out_v, sem0, sem1):
        wid = lax.axis_index("s") * ncores + lax.axis_index("c")
        base = wid * rpw
        pltpu.sync_copy(gtext_hbm.at[pl.ds(2 * base, nchunk)], gidx_v)
        pltpu.sync_copy(hbits_hbm.at[pl.ds(2 * base * WPC, nchunk * WPC)],
                        bits_v.at[pl.ds(0, nchunk * WPC)])
        bufs = (rows0, rows1)
        sems = (sem0, sem1)

        def issue(b, t):
            pltpu.async_copy(emb_hbm.at[gidx_v.at[2 * b]],
                             bufs[t].at[pl.ds(0, HALF)], sems[t])
            pltpu.async_copy(emb_hbm.at[gidx_v.at[2 * b + 1]],
                             bufs[t].at[pl.ds(HALF, HALF)], sems[t])

        def drain(t):
            pltpu.make_async_copy(emb_hbm.at[pl.ds(0, L)], bufs[t], sems[t]).wait()

        def consume(b, t):
            drain(t)
            buf = bufs[t]
            zero = jnp.zeros((16,), jnp.float32)

            def make_rbody(h):
                def rbody(rr, acc):
                    w = bits_v[pl.ds((2 * b + h) * WPC + (rr >> 5), 16)][0]
                    off = (lax.shift_right_logical(w, rr & 31) & 1) * D
                    r = h * HALF + rr
                    return tuple(acc[d] + buf[r, pl.ds(off + d * 16, 16)]
                                 for d in range(NVREG))
                return rbody

            acc = lax.fori_loop(0, HALF, make_rbody(0), (zero,) * NVREG,
                                unroll=8)
            acc = lax.fori_loop(0, HALF, make_rbody(1), acc, unroll=8)
            inv = jnp.float32(1.0 / L)
            for d in range(NVREG):
                out_v[b, pl.ds(d * 16, 16)] = acc[d] * inv

        issue(0, 0)

        def outer(i, carry):
            for t in range(2):
                b = 2 * i + t

                @pl.when(b + 1 < rpw)
                def _():
                    issue(b + 1, (t + 1) % 2)

                consume(b, t)
            return carry

        lax.fori_loop(0, rpw // 2, outer, 0)
        pltpu.sync_copy(out_v, out_hbm.at[pl.ds(base, rpw)])

    return pool(gtext, hbits, emb2)


KTR = 8192
NTR = 62
M = KTR * NTR


def _tr_body(xa_ref, xb_ref, o_ref):
    o_ref[:, 0:D] = xa_ref[...].T
    o_ref[:, D:2 * D] = xb_ref[...].T


def _transpose_tc(embT):
    return pl.pallas_call(
        _tr_body,
        grid=(NTR,),
        in_specs=[pl.BlockSpec((D, KTR), lambda i: (0, i)),
                  pl.BlockSpec((D, KTR),
                               lambda i: (0, jnp.minimum(i + NTR, 122)))],
        out_specs=pl.BlockSpec((KTR, 2 * D), lambda i: (i, 0)),
        out_shape=jax.ShapeDtypeStruct((M, 2 * D), jnp.float32),
    )(embT, embT)


def _mlp_body(x_ref, w1_ref, b1_ref, w2_ref, b2_ref, o_ref):
    x = x_ref[...]
    h = lax.dot_general(x, w1_ref[...], (((1,), (1,)), ((), ())),
                        preferred_element_type=jnp.float32)
    h = jnp.maximum(h + b1_ref[...], 0.0)
    o = lax.dot_general(h, w2_ref[...], (((1,), (1,)), ((), ())),
                        preferred_element_type=jnp.float32)
    o_ref[...] = o + b2_ref[...]


def _mlp_tc(pooled, W1, b1, W2, b2):
    blk = 512
    return pl.pallas_call(
        _mlp_body,
        grid=(B // blk,),
        in_specs=[
            pl.BlockSpec((blk, D), lambda i: (i, 0)),
            pl.BlockSpec((H, D), lambda i: (0, 0)),
            pl.BlockSpec((1, H), lambda i: (0, 0)),
            pl.BlockSpec((O, H), lambda i: (0, 0)),
            pl.BlockSpec((1, O), lambda i: (0, 0)),
        ],
        out_specs=pl.BlockSpec((blk, O), lambda i: (i, 0)),
        out_shape=jax.ShapeDtypeStruct((B, O), jnp.float32),
    )(pooled, W1, b1.reshape(1, H), W2, b2.reshape(1, O))


def kernel(text, emb, W1, b1, W2, b2):
    ti = text.astype(jnp.int32)
    hi = (ti >= M).astype(jnp.int32)
    gtext = (ti - hi * M).reshape(2 * B, HALF)
    par = hi.reshape(2 * B, HALF)
    parp = jnp.pad(par, ((0, 0), (0, 28))).reshape(2 * B, WPC, 32)
    shifts = jnp.arange(32, dtype=jnp.int32)[None, None, :]
    hbits = (parp << shifts).sum(axis=2, dtype=jnp.int32).reshape(-1)
    emb2 = _transpose_tc(emb.T)
    pooled = _pool_sc(gtext, hbits, emb2)
    return _mlp_tc(pooled, W1, b1, W2, b2)


# speedup vs baseline: 1.5597x; 1.0280x over previous
import functools

import jax
import jax.numpy as jnp
from jax import lax
from jax.experimental import pallas as pl
from jax.experimental.pallas import tpu as pltpu
from jax.experimental.pallas import tpu_sc as plsc

B = 4096
L = 200
D = 64
H = 128
O = 10
HALF = L // 2
NVREG = D // 16
WPC = 4


def _pool_sc(gtext, hbits, emb2):
    info = plsc.get_sparse_core_info()
    ncores = info.num_cores
    nw = ncores * info.num_subcores
    rpw = B // nw
    nchunk = 2 * rpw
    mesh = plsc.VectorSubcoreMesh(core_axis_name="c", subcore_axis_name="s")

    @functools.partial(
        pl.kernel,
        out_type=jax.ShapeDtypeStruct((B, D), jnp.float32),
        mesh=mesh,
        scratch_types=[
            pltpu.VMEM((nchunk, HALF), jnp.int32),
            pltpu.VMEM((nchunk * WPC + 16,), jnp.int32),
            pltpu.VMEM((L, 2 * D), jnp.float32),
            pltpu.VMEM((L, 2 * D), jnp.float32),
            pltpu.VMEM((rpw, D), jnp.float32),
            pltpu.SemaphoreType.DMA,
            pltpu.SemaphoreType.DMA,
        ],
    )
    def pool(gtext_hbm, hbits_hbm, emb_hbm, out_hbm,
             gidx_v, bits_v, rows0, rows1, out_v, sem0, sem1):
        wid = lax.axis_index("s") * ncores + lax.axis_index("c")
        base = wid * rpw
        pltpu.sync_copy(gtext_hbm.at[pl.ds(2 * base, nchunk)], gidx_v)
        pltpu.sync_copy(hbits_hbm.at[pl.ds(2 * base * WPC, nchunk * WPC)],
                        bits_v.at[pl.ds(0, nchunk * WPC)])
        bufs = (rows0, rows1)
        sems = (sem0, sem1)

        def issue(b, t):
            pltpu.async_copy(emb_hbm.at[gidx_v.at[2 * b]],
                             bufs[t].at[pl.ds(0, HALF)], sems[t])
            pltpu.async_copy(emb_hbm.at[gidx_v.at[2 * b + 1]],
                             bufs[t].at[pl.ds(HALF, HALF)], sems[t])

        def drain(t):
            pltpu.make_async_copy(emb_hbm.at[pl.ds(0, L)], bufs[t], sems[t]).wait()

        def consume(b, t):
            drain(t)
            buf = bufs[t]
            zero = jnp.zeros((16,), jnp.float32)

            def make_rbody(h):
                def rbody(rr, acc):
                    w = bits_v[pl.ds((2 * b + h) * WPC + (rr >> 5), 16)][0]
                    off = (lax.shift_right_logical(w, rr & 31) & 1) * D
                    r = h * HALF + rr
                    return tuple(acc[d] + buf[r, pl.ds(off + d * 16, 16)]
                                 for d in range(NVREG))
                return rbody

            acc = lax.fori_loop(0, HALF, make_rbody(0), (zero,) * NVREG,
                                unroll=8)
            acc = lax.fori_loop(0, HALF, make_rbody(1), acc, unroll=8)
            inv = jnp.float32(1.0 / L)
            for d in range(NVREG):
                out_v[b, pl.ds(d * 16, 16)] = acc[d] * inv

        issue(0, 0)

        def outer(i, carry):
            for t in range(2):
                b = 2 * i + t

                @pl.when(b + 1 < rpw)
                def _():
                    issue(b + 1, (t + 1) % 2)

                consume(b, t)
            return carry

        lax.fori_loop(0, rpw // 2, outer, 0)
        pltpu.sync_copy(out_v, out_hbm.at[pl.ds(base, rpw)])

    return pool(gtext, hbits, emb2)


KTR = 16384
NTR = 31
M = KTR * NTR


def _tr_body(xa_ref, xb_ref, o_ref):
    o_ref[:, 0:D] = xa_ref[...].T
    o_ref[:, D:2 * D] = xb_ref[...].T


def _transpose_tc(embT):
    return pl.pallas_call(
        _tr_body,
        grid=(NTR,),
        in_specs=[pl.BlockSpec((D, KTR), lambda i: (0, i)),
                  pl.BlockSpec((D, KTR),
                               lambda i: (0, jnp.minimum(i + NTR, 61)))],
        out_specs=pl.BlockSpec((KTR, 2 * D), lambda i: (i, 0)),
        out_shape=jax.ShapeDtypeStruct((M, 2 * D), jnp.float32),
    )(embT, embT)


def _mlp_body(x_ref, w1_ref, b1_ref, w2_ref, b2_ref, o_ref):
    x = x_ref[...]
    h = lax.dot_general(x, w1_ref[...], (((1,), (1,)), ((), ())),
                        preferred_element_type=jnp.float32)
    h = jnp.maximum(h + b1_ref[...], 0.0)
    o = lax.dot_general(h, w2_ref[...], (((1,), (1,)), ((), ())),
                        preferred_element_type=jnp.float32)
    o_ref[...] = o + b2_ref[...]


def _mlp_tc(pooled, W1, b1, W2, b2):
    blk = 512
    return pl.pallas_call(
        _mlp_body,
        grid=(B // blk,),
        in_specs=[
            pl.BlockSpec((blk, D), lambda i: (i, 0)),
            pl.BlockSpec((H, D), lambda i: (0, 0)),
            pl.BlockSpec((1, H), lambda i: (0, 0)),
            pl.BlockSpec((O, H), lambda i: (0, 0)),
            pl.BlockSpec((1, O), lambda i: (0, 0)),
        ],
        out_specs=pl.BlockSpec((blk, O), lambda i: (i, 0)),
        out_shape=jax.ShapeDtypeStruct((B, O), jnp.float32),
    )(pooled, W1, b1.reshape(1, H), W2, b2.reshape(1, O))


def kernel(text, emb, W1, b1, W2, b2):
    ti = text.astype(jnp.int32)
    hi = (ti >= M).astype(jnp.int32)
    gtext = (ti - hi * M).reshape(2 * B, HALF)
    par = hi.reshape(2 * B, HALF)
    parp = jnp.pad(par, ((0, 0), (0, 28))).reshape(2 * B, WPC, 32)
    shifts = jnp.arange(32, dtype=jnp.int32)[None, None, :]
    hbits = (parp << shifts).sum(axis=2, dtype=jnp.int32).reshape(-1)
    emb2 = _transpose_tc(emb.T)
    pooled = _pool_sc(gtext, hbits, emb2)
    return _mlp_tc(pooled, W1, b1, W2, b2)
